# Initial kernel scaffold; baseline (speedup 1.0000x reference)
import jax
import jax.numpy as jnp
from jax.experimental import pallas as pl


def kernel(x, hyperedge_index, W1, b1, gamma, beta, W3, b3):
    raise NotImplementedError("write your pallas kernel here")


# speedup vs baseline: 8.7701x; 8.7701x over previous
import jax
import jax.numpy as jnp
from jax import lax
from jax.experimental import pallas as pl
from jax.experimental.pallas import tpu as pltpu
from jax.experimental.pallas import tpu_sc as plsc

N = 10000
NP = 10112
D = 128
E = 320000
NC = 2
NS = 16
L = 16
EPC = E // NC
EPW = EPC // NS
CH = 128
NFULL = EPW // CH
TAIL = EPW - NFULL * CH
RPT = NP // NS


def _fill_const(ref, rows, cols, value):
    vec = jnp.full((L,), value, jnp.float32)

    def body(k, _):
        i = k // (cols // L)
        j = k % (cols // L)
        ref[i, pl.ds(j * L, L)] = vec
        return 0

    lax.fori_loop(0, rows * (cols // L), body, 0)


def _row_blocks(total):
    out = []
    off = 0
    while off < total:
        n = min(CH, total - off)
        out.append((off, n))
        off += n
    return out


def _zero_own_rows(zeros, acc_sh, r0):
    for off, n in _row_blocks(RPT):
        pltpu.sync_copy(zeros.at[pl.ds(0, n)], acc_sh.at[pl.ds(r0 + off, n)])


def _writeback_own_rows(acc_sh, out, cid, r0):
    for off, n in _row_blocks(RPT):
        pltpu.sync_copy(acc_sh.at[pl.ds(r0 + off, n)],
                        out.at[cid, pl.ds(r0 + off, n)])


def _sc_stage_body(table, gidx, sidx, out,
                   acc_sh, idxg, idxs, rows, idxg_t, idxs_t, rows_t,
                   zeros, gsem):
    cid = lax.axis_index("c")
    sid = lax.axis_index("s")

    _fill_const(zeros, CH, D, 0.0)
    r0 = pl.multiple_of(sid * RPT, 8)
    _zero_own_rows(zeros, acc_sh, r0)
    plsc.subcore_barrier()

    base = cid * EPC + sid * EPW

    def chunk(i, _):
        off = base + i * CH
        pltpu.sync_copy(gidx.at[pl.ds(off, CH)], idxg)
        pltpu.sync_copy(sidx.at[pl.ds(off, CH)], idxs)
        pltpu.async_copy(table.at[idxg], rows, gsem).wait()
        pltpu.sync_copy(rows, acc_sh.at[idxs], add=True)
        return 0

    lax.fori_loop(0, NFULL, chunk, 0)

    if TAIL:
        off = base + NFULL * CH
        pltpu.sync_copy(gidx.at[pl.ds(off, TAIL)], idxg_t)
        pltpu.sync_copy(sidx.at[pl.ds(off, TAIL)], idxs_t)
        pltpu.async_copy(table.at[idxg_t], rows_t, gsem).wait()
        pltpu.sync_copy(rows_t, acc_sh.at[idxs_t], add=True)

    plsc.subcore_barrier()
    _writeback_own_rows(acc_sh, out, cid, r0)


_sc_mesh = plsc.VectorSubcoreMesh(core_axis_name="c", subcore_axis_name="s",
                                  num_cores=NC, num_subcores=NS)

_sc_stage = pl.kernel(
    _sc_stage_body,
    out_type=jax.ShapeDtypeStruct((NC, NP, D), jnp.float32),
    mesh=_sc_mesh,
    scratch_types=(
        pltpu.VMEM_SHARED((NP, D), jnp.float32),
        pltpu.VMEM((CH,), jnp.int32),
        pltpu.VMEM((CH,), jnp.int32),
        pltpu.VMEM((CH, D), jnp.float32),
        pltpu.VMEM((TAIL,), jnp.int32),
        pltpu.VMEM((TAIL,), jnp.int32),
        pltpu.VMEM((TAIL, D), jnp.float32),
        pltpu.VMEM((CH, D), jnp.float32),
        pltpu.SemaphoreType.DMA,
    ),
)


def _sc_counts_body(gidx, sidx, cntg_out, cnts_out,
                    acc_sh, idx, idx_t, zeros, ones, ones_t):
    cid = lax.axis_index("c")
    sid = lax.axis_index("s")

    _fill_const(zeros, CH, D, 0.0)
    _fill_const(ones, CH, D, 1.0)
    _fill_const(ones_t, TAIL, D, 1.0)
    r0 = pl.multiple_of(sid * RPT, 8)
    base = cid * EPC + sid * EPW

    for idx_arr, out in ((gidx, cntg_out), (sidx, cnts_out)):
        _zero_own_rows(zeros, acc_sh, r0)
        plsc.subcore_barrier()

        def chunk(i, _):
            off = base + i * CH
            pltpu.sync_copy(idx_arr.at[pl.ds(off, CH)], idx)
            pltpu.sync_copy(ones, acc_sh.at[idx], add=True)
            return 0

        lax.fori_loop(0, NFULL, chunk, 0)
        if TAIL:
            off = base + NFULL * CH
            pltpu.sync_copy(idx_arr.at[pl.ds(off, TAIL)], idx_t)
            pltpu.sync_copy(ones_t, acc_sh.at[idx_t], add=True)

        plsc.subcore_barrier()
        _writeback_own_rows(acc_sh, out, cid, r0)
        plsc.subcore_barrier()


_sc_counts = pl.kernel(
    _sc_counts_body,
    out_type=(jax.ShapeDtypeStruct((NC, NP, D), jnp.float32),
              jax.ShapeDtypeStruct((NC, NP, D), jnp.float32)),
    mesh=_sc_mesh,
    scratch_types=(
        pltpu.VMEM_SHARED((NP, D), jnp.float32),
        pltpu.VMEM((CH,), jnp.int32),
        pltpu.VMEM((TAIL,), jnp.int32),
        pltpu.VMEM((CH, D), jnp.float32),
        pltpu.VMEM((CH, D), jnp.float32),
        pltpu.VMEM((TAIL, D), jnp.float32),
    ),
)


_RB = 1000
_G = N // _RB


def _mm_body(x_ref, w_ref, o_ref):
    o_ref[...] = jnp.dot(x_ref[...], w_ref[...],
                         preferred_element_type=jnp.float32)


def _tc_matmul(x, w):
    return pl.pallas_call(
        _mm_body,
        grid=(_G,),
        in_specs=[pl.BlockSpec((_RB, D), lambda i: (i, 0)),
                  pl.BlockSpec((D, D), lambda i: (0, 0))],
        out_specs=pl.BlockSpec((_RB, D), lambda i: (i, 0)),
        out_shape=jax.ShapeDtypeStruct((N, D), jnp.float32),
    )(x, w)


def _inv_counts(c_ref):
    c = c_ref[0, :, 0:1] + c_ref[1, :, 0:1]
    return jnp.where(c > 0, 1.0 / c, 0.0)


def _comb_body(p_ref, c_ref, o_ref):
    o_ref[...] = (p_ref[0] + p_ref[1]) * _inv_counts(c_ref)


def _tc_combine(p, c):
    return pl.pallas_call(
        _comb_body,
        grid=(_G,),
        in_specs=[pl.BlockSpec((NC, _RB, D), lambda i: (0, i, 0)),
                  pl.BlockSpec((NC, _RB, D), lambda i: (0, i, 0))],
        out_specs=pl.BlockSpec((_RB, D), lambda i: (i, 0)),
        out_shape=jax.ShapeDtypeStruct((N, D), jnp.float32),
    )(p, c)


def _mid_body(p_ref, c_ref, b_ref, g_ref, be_ref, w_ref, o_ref):
    h = (p_ref[0] + p_ref[1]) * _inv_counts(c_ref) + b_ref[...]
    mu = jnp.mean(h, axis=-1, keepdims=True)
    var = jnp.mean((h - mu) * (h - mu), axis=-1, keepdims=True)
    h = (h - mu) * lax.rsqrt(var + 1e-5) * g_ref[...] + be_ref[...]
    h = jnp.where(h >= 0, h, 0.01 * h)
    o_ref[...] = jnp.dot(h, w_ref[...], preferred_element_type=jnp.float32)


def _tc_mid(p, c, b1, gamma, beta, w3):
    vec = pl.BlockSpec((1, D), lambda i: (0, 0))
    return pl.pallas_call(
        _mid_body,
        grid=(_G,),
        in_specs=[pl.BlockSpec((NC, _RB, D), lambda i: (0, i, 0)),
                  pl.BlockSpec((NC, _RB, D), lambda i: (0, i, 0)),
                  vec, vec, vec,
                  pl.BlockSpec((D, D), lambda i: (0, 0))],
        out_specs=pl.BlockSpec((_RB, D), lambda i: (i, 0)),
        out_shape=jax.ShapeDtypeStruct((N, D), jnp.float32),
    )(p, c, b1.reshape(1, D), gamma.reshape(1, D), beta.reshape(1, D), w3)


def _final_body(p_ref, c_ref, b_ref, o_ref):
    o_ref[...] = (p_ref[0] + p_ref[1]) * _inv_counts(c_ref) + b_ref[...]


def _tc_final(p, c, b3):
    return pl.pallas_call(
        _final_body,
        grid=(_G,),
        in_specs=[pl.BlockSpec((NC, _RB, D), lambda i: (0, i, 0)),
                  pl.BlockSpec((NC, _RB, D), lambda i: (0, i, 0)),
                  pl.BlockSpec((1, D), lambda i: (0, 0))],
        out_specs=pl.BlockSpec((_RB, D), lambda i: (i, 0)),
        out_shape=jax.ShapeDtypeStruct((N, D), jnp.float32),
    )(p, c, b3.reshape(1, D))


@jax.jit
def kernel(x, hyperedge_index, W1, b1, gamma, beta, W3, b3):
    node_idx = hyperedge_index[0]
    he_idx = hyperedge_index[1]

    cnt_node, cnt_he = _sc_counts(node_idx, he_idx)
    x1 = _tc_matmul(x, W1)
    e_p = _sc_stage(x1, node_idx, he_idx)
    e1 = _tc_combine(e_p, cnt_he)
    o_p = _sc_stage(e1, he_idx, node_idx)
    x2 = _tc_mid(o_p, cnt_node, b1, gamma, beta, W3)
    e2_p = _sc_stage(x2, node_idx, he_idx)
    e2 = _tc_combine(e2_p, cnt_he)
    o2_p = _sc_stage(e2, he_idx, node_idx)
    return _tc_final(o2_p, cnt_node, b3)


# speedup vs baseline: 12.6261x; 1.4397x over previous
import jax
import jax.numpy as jnp
from jax import lax
from jax.experimental import pallas as pl
from jax.experimental.pallas import tpu as pltpu
from jax.experimental.pallas import tpu_sc as plsc

N = 10000
NP = 10112
D = 128
E = 320000
NC = 2
NS = 16
L = 16
EPC = E // NC
EPW = EPC // NS
CH = 128
NFULL = EPW // CH
TAIL = EPW - NFULL * CH
RPT = NP // NS


def _fill_const(ref, rows, cols, value):
    vec = jnp.full((L,), value, jnp.float32)

    def body(k, _):
        i = k // (cols // L)
        j = k % (cols // L)
        ref[i, pl.ds(j * L, L)] = vec
        return 0

    lax.fori_loop(0, rows * (cols // L), body, 0)


def _row_blocks(total, chunk=CH):
    out = []
    off = 0
    while off < total:
        n = min(chunk, total - off)
        out.append((off, n))
        off += n
    return out


ZCH = 64


def _zero_own_rows(zeros, acc_sh, r0):
    for off, n in _row_blocks(RPT, ZCH):
        pltpu.sync_copy(zeros.at[pl.ds(0, n)], acc_sh.at[pl.ds(r0 + off, n)])


def _writeback_own_rows(acc_sh, out, cid, r0):
    for off, n in _row_blocks(RPT):
        pltpu.sync_copy(acc_sh.at[pl.ds(r0 + off, n)],
                        out.at[cid, pl.ds(r0 + off, n)])


def _sc_stage_body(table, gidx, sidx, out,
                   acc_sh, idxg, idxs, rows, idxg_t, idxs_t, rows_t,
                   zeros, gsem):
    cid = lax.axis_index("c")
    sid = lax.axis_index("s")

    _fill_const(zeros, ZCH, D, 0.0)
    r0 = pl.multiple_of(sid * RPT, 8)
    _zero_own_rows(zeros, acc_sh, r0)
    plsc.subcore_barrier()

    base = cid * EPC + sid * EPW

    def load_and_gather(i, b):
        off = base + i * CH
        pltpu.sync_copy(gidx.at[pl.ds(off, CH)], idxg.at[b])
        pltpu.sync_copy(sidx.at[pl.ds(off, CH)], idxs.at[b])
        pltpu.make_async_copy(table.at[idxg.at[b]], rows.at[b], gsem).start()

    load_and_gather(0, 0)

    def chunk(i, _):
        b = lax.rem(i, 2)

        @pl.when(i + 1 < NFULL)
        def _():
            load_and_gather(i + 1, 1 - b)

        pltpu.make_async_copy(table.at[idxg.at[b]], rows.at[b], gsem).wait()
        pltpu.sync_copy(rows.at[b], acc_sh.at[idxs.at[b]], add=True)
        return 0

    lax.fori_loop(0, NFULL, chunk, 0)

    if TAIL:
        off = base + NFULL * CH
        pltpu.sync_copy(gidx.at[pl.ds(off, TAIL)], idxg_t)
        pltpu.sync_copy(sidx.at[pl.ds(off, TAIL)], idxs_t)
        pltpu.async_copy(table.at[idxg_t], rows_t, gsem).wait()
        pltpu.sync_copy(rows_t, acc_sh.at[idxs_t], add=True)

    plsc.subcore_barrier()
    _writeback_own_rows(acc_sh, out, cid, r0)


_sc_mesh = plsc.VectorSubcoreMesh(core_axis_name="c", subcore_axis_name="s",
                                  num_cores=NC, num_subcores=NS)

_sc_stage = pl.kernel(
    _sc_stage_body,
    out_type=jax.ShapeDtypeStruct((NC, NP, D), jnp.float32),
    mesh=_sc_mesh,
    scratch_types=(
        pltpu.VMEM_SHARED((NP, D), jnp.float32),
        pltpu.VMEM((2, CH), jnp.int32),
        pltpu.VMEM((2, CH), jnp.int32),
        pltpu.VMEM((2, CH, D), jnp.float32),
        pltpu.VMEM((TAIL,), jnp.int32),
        pltpu.VMEM((TAIL,), jnp.int32),
        pltpu.VMEM((TAIL, D), jnp.float32),
        pltpu.VMEM((ZCH, D), jnp.float32),
        pltpu.SemaphoreType.DMA,
    ),
)


def _sc_counts_body(gidx, sidx, cntg_out, cnts_out,
                    acc_sh, idx, idx_t, zeros, ones, ones_t):
    cid = lax.axis_index("c")
    sid = lax.axis_index("s")

    _fill_const(zeros, ZCH, D, 0.0)
    _fill_const(ones, CH, D, 1.0)
    _fill_const(ones_t, TAIL, D, 1.0)
    r0 = pl.multiple_of(sid * RPT, 8)
    base = cid * EPC + sid * EPW

    for idx_arr, out in ((gidx, cntg_out), (sidx, cnts_out)):
        _zero_own_rows(zeros, acc_sh, r0)
        plsc.subcore_barrier()

        def chunk(i, _):
            off = base + i * CH
            pltpu.sync_copy(idx_arr.at[pl.ds(off, CH)], idx)
            pltpu.sync_copy(ones, acc_sh.at[idx], add=True)
            return 0

        lax.fori_loop(0, NFULL, chunk, 0)
        if TAIL:
            off = base + NFULL * CH
            pltpu.sync_copy(idx_arr.at[pl.ds(off, TAIL)], idx_t)
            pltpu.sync_copy(ones_t, acc_sh.at[idx_t], add=True)

        plsc.subcore_barrier()
        _writeback_own_rows(acc_sh, out, cid, r0)
        plsc.subcore_barrier()


_sc_counts = pl.kernel(
    _sc_counts_body,
    out_type=(jax.ShapeDtypeStruct((NC, NP, D), jnp.float32),
              jax.ShapeDtypeStruct((NC, NP, D), jnp.float32)),
    mesh=_sc_mesh,
    scratch_types=(
        pltpu.VMEM_SHARED((NP, D), jnp.float32),
        pltpu.VMEM((CH,), jnp.int32),
        pltpu.VMEM((TAIL,), jnp.int32),
        pltpu.VMEM((ZCH, D), jnp.float32),
        pltpu.VMEM((CH, D), jnp.float32),
        pltpu.VMEM((TAIL, D), jnp.float32),
    ),
)


_RB = 1000
_G = N // _RB


def _mm_body(x_ref, w_ref, o_ref):
    o_ref[...] = jnp.dot(x_ref[...], w_ref[...],
                         preferred_element_type=jnp.float32)


def _tc_matmul(x, w):
    return pl.pallas_call(
        _mm_body,
        grid=(_G,),
        in_specs=[pl.BlockSpec((_RB, D), lambda i: (i, 0)),
                  pl.BlockSpec((D, D), lambda i: (0, 0))],
        out_specs=pl.BlockSpec((_RB, D), lambda i: (i, 0)),
        out_shape=jax.ShapeDtypeStruct((N, D), jnp.float32),
    )(x, w)


def _inv_counts(c_ref):
    c = c_ref[0, :, 0:1] + c_ref[1, :, 0:1]
    return jnp.where(c > 0, 1.0 / c, 0.0)


def _comb_body(p_ref, c_ref, o_ref):
    o_ref[...] = (p_ref[0] + p_ref[1]) * _inv_counts(c_ref)


def _tc_combine(p, c):
    return pl.pallas_call(
        _comb_body,
        grid=(_G,),
        in_specs=[pl.BlockSpec((NC, _RB, D), lambda i: (0, i, 0)),
                  pl.BlockSpec((NC, _RB, D), lambda i: (0, i, 0))],
        out_specs=pl.BlockSpec((_RB, D), lambda i: (i, 0)),
        out_shape=jax.ShapeDtypeStruct((N, D), jnp.float32),
    )(p, c)


def _mid_body(p_ref, c_ref, b_ref, g_ref, be_ref, w_ref, o_ref):
    h = (p_ref[0] + p_ref[1]) * _inv_counts(c_ref) + b_ref[...]
    mu = jnp.mean(h, axis=-1, keepdims=True)
    var = jnp.mean((h - mu) * (h - mu), axis=-1, keepdims=True)
    h = (h - mu) * lax.rsqrt(var + 1e-5) * g_ref[...] + be_ref[...]
    h = jnp.where(h >= 0, h, 0.01 * h)
    o_ref[...] = jnp.dot(h, w_ref[...], preferred_element_type=jnp.float32)


def _tc_mid(p, c, b1, gamma, beta, w3):
    vec = pl.BlockSpec((1, D), lambda i: (0, 0))
    return pl.pallas_call(
        _mid_body,
        grid=(_G,),
        in_specs=[pl.BlockSpec((NC, _RB, D), lambda i: (0, i, 0)),
                  pl.BlockSpec((NC, _RB, D), lambda i: (0, i, 0)),
                  vec, vec, vec,
                  pl.BlockSpec((D, D), lambda i: (0, 0))],
        out_specs=pl.BlockSpec((_RB, D), lambda i: (i, 0)),
        out_shape=jax.ShapeDtypeStruct((N, D), jnp.float32),
    )(p, c, b1.reshape(1, D), gamma.reshape(1, D), beta.reshape(1, D), w3)


def _final_body(p_ref, c_ref, b_ref, o_ref):
    o_ref[...] = (p_ref[0] + p_ref[1]) * _inv_counts(c_ref) + b_ref[...]


def _tc_final(p, c, b3):
    return pl.pallas_call(
        _final_body,
        grid=(_G,),
        in_specs=[pl.BlockSpec((NC, _RB, D), lambda i: (0, i, 0)),
                  pl.BlockSpec((NC, _RB, D), lambda i: (0, i, 0)),
                  pl.BlockSpec((1, D), lambda i: (0, 0))],
        out_specs=pl.BlockSpec((_RB, D), lambda i: (i, 0)),
        out_shape=jax.ShapeDtypeStruct((N, D), jnp.float32),
    )(p, c, b3.reshape(1, D))


@jax.jit
def kernel(x, hyperedge_index, W1, b1, gamma, beta, W3, b3):
    node_idx = hyperedge_index[0]
    he_idx = hyperedge_index[1]

    cnt_node, cnt_he = _sc_counts(node_idx, he_idx)
    x1 = _tc_matmul(x, W1)
    e_p = _sc_stage(x1, node_idx, he_idx)
    e1 = _tc_combine(e_p, cnt_he)
    o_p = _sc_stage(e1, he_idx, node_idx)
    x2 = _tc_mid(o_p, cnt_node, b1, gamma, beta, W3)
    e2_p = _sc_stage(x2, node_idx, he_idx)
    e2 = _tc_combine(e2_p, cnt_he)
    o2_p = _sc_stage(e2, he_idx, node_idx)
    return _tc_final(o2_p, cnt_node, b3)


# speedup vs baseline: 15.7198x; 1.2450x over previous
import jax
import jax.numpy as jnp
from jax import lax
from jax.experimental import pallas as pl
from jax.experimental.pallas import tpu as pltpu
from jax.experimental.pallas import tpu_sc as plsc

N = 10000
NP = 10112
D = 128
E = 320000
NC = 2
NS = 16
L = 16
EPC = E // NC
EPW = EPC // NS
CH = 128
NFULL = EPW // CH
TAIL = EPW - NFULL * CH
RPT = NP // NS
ZCH = 64


def _fill_const(ref, rows, cols, value):
    vec = jnp.full((L,), value, jnp.float32)

    def body(k, _):
        i = k // (cols // L)
        j = k % (cols // L)
        ref[i, pl.ds(j * L, L)] = vec
        return 0

    lax.fori_loop(0, rows * (cols // L), body, 0)


def _row_blocks(total, chunk=CH):
    out = []
    off = 0
    while off < total:
        n = min(chunk, total - off)
        out.append((off, n))
        off += n
    return out


def _zero_own_rows(zeros, acc_sh, r0):
    for off, n in _row_blocks(RPT, ZCH):
        pltpu.sync_copy(zeros.at[pl.ds(0, n)], acc_sh.at[pl.ds(r0 + off, n)])


def _writeback_own_rows(acc_sh, out, cid, r0):
    for off, n in _row_blocks(RPT):
        pltpu.sync_copy(acc_sh.at[pl.ds(r0 + off, n)],
                        out.at[cid, pl.ds(r0 + off, n)])


def _sc_stage_body(table, gidx, sidx, out,
                   acc_sh, idxg, idxs, rows, idxg_t, idxs_t, rows_t,
                   zeros, isem, gsem, ssem):
    cid = lax.axis_index("c")
    sid = lax.axis_index("s")

    _fill_const(zeros, ZCH, D, 0.0)
    r0 = pl.multiple_of(sid * RPT, 8)
    _zero_own_rows(zeros, acc_sh, r0)
    plsc.subcore_barrier()

    base = cid * EPC + sid * EPW

    def idx_start(i, sl):
        off = base + i * CH
        pltpu.async_copy(gidx.at[pl.ds(off, CH)], idxg.at[sl], isem)
        pltpu.async_copy(sidx.at[pl.ds(off, CH)], idxs.at[sl], isem)

    def idx_wait(i, sl):
        off = base + i * CH
        pltpu.make_async_copy(gidx.at[pl.ds(off, CH)], idxg.at[sl],
                              isem).wait()
        pltpu.make_async_copy(sidx.at[pl.ds(off, CH)], idxs.at[sl],
                              isem).wait()

    def gather_start(sl, b):
        pltpu.async_copy(table.at[idxg.at[sl]], rows.at[b], gsem)

    def gather_wait(sl, b):
        pltpu.make_async_copy(table.at[idxg.at[sl]], rows.at[b], gsem).wait()

    def scatter_start(sl, b):
        pltpu.async_copy(rows.at[b], acc_sh.at[idxs.at[sl]], ssem, add=True)

    def scatter_wait(sl, b):
        pltpu.make_async_copy(rows.at[b], acc_sh.at[idxs.at[sl]], ssem).wait()

    idx_start(0, 0)
    idx_start(1, 1)
    idx_wait(0, 0)
    gather_start(0, 0)

    def body(i, _):
        sl = lax.rem(i, 3)
        b = lax.rem(i, 2)
        sln = lax.rem(i + 1, 3)
        bn = lax.rem(i + 1, 2)

        @pl.when(i >= 1)
        def _():
            scatter_wait(lax.rem(i - 1, 3), bn)

        @pl.when(i + 2 < NFULL)
        def _():
            idx_start(i + 2, lax.rem(i + 2, 3))

        @pl.when(i + 1 < NFULL)
        def _():
            idx_wait(i + 1, sln)
            gather_start(sln, bn)

        gather_wait(sl, b)
        scatter_start(sl, b)
        return 0

    lax.fori_loop(0, NFULL, body, 0)
    scatter_wait(lax.rem(NFULL - 1, 3), lax.rem(NFULL - 1, 2))

    if TAIL:
        off = base + NFULL * CH
        pltpu.sync_copy(gidx.at[pl.ds(off, TAIL)], idxg_t)
        pltpu.sync_copy(sidx.at[pl.ds(off, TAIL)], idxs_t)
        pltpu.async_copy(table.at[idxg_t], rows_t, gsem).wait()
        pltpu.sync_copy(rows_t, acc_sh.at[idxs_t], add=True)

    plsc.subcore_barrier()
    _writeback_own_rows(acc_sh, out, cid, r0)


_sc_mesh = plsc.VectorSubcoreMesh(core_axis_name="c", subcore_axis_name="s",
                                  num_cores=NC, num_subcores=NS)

_sc_stage = pl.kernel(
    _sc_stage_body,
    out_type=jax.ShapeDtypeStruct((NC, NP, D), jnp.float32),
    mesh=_sc_mesh,
    scratch_types=(
        pltpu.VMEM_SHARED((NP, D), jnp.float32),
        pltpu.VMEM((3, CH), jnp.int32),
        pltpu.VMEM((3, CH), jnp.int32),
        pltpu.VMEM((2, CH, D), jnp.float32),
        pltpu.VMEM((TAIL,), jnp.int32),
        pltpu.VMEM((TAIL,), jnp.int32),
        pltpu.VMEM((TAIL, D), jnp.float32),
        pltpu.VMEM((ZCH, D), jnp.float32),
        pltpu.SemaphoreType.DMA,
        pltpu.SemaphoreType.DMA,
        pltpu.SemaphoreType.DMA,
    ),
)


def _sc_counts_body(gidx, sidx, cntg_out, cnts_out,
                    acc_sh, idx, idx_t, zeros, ones, ones_t):
    cid = lax.axis_index("c")
    sid = lax.axis_index("s")

    _fill_const(zeros, ZCH, D, 0.0)
    _fill_const(ones, CH, D, 1.0)
    _fill_const(ones_t, TAIL, D, 1.0)
    r0 = pl.multiple_of(sid * RPT, 8)
    base = cid * EPC + sid * EPW

    for idx_arr, out in ((gidx, cntg_out), (sidx, cnts_out)):
        _zero_own_rows(zeros, acc_sh, r0)
        plsc.subcore_barrier()

        def chunk(i, _):
            off = base + i * CH
            pltpu.sync_copy(idx_arr.at[pl.ds(off, CH)], idx)
            pltpu.sync_copy(ones, acc_sh.at[idx], add=True)
            return 0

        lax.fori_loop(0, NFULL, chunk, 0)
        if TAIL:
            off = base + NFULL * CH
            pltpu.sync_copy(idx_arr.at[pl.ds(off, TAIL)], idx_t)
            pltpu.sync_copy(ones_t, acc_sh.at[idx_t], add=True)

        plsc.subcore_barrier()
        _writeback_own_rows(acc_sh, out, cid, r0)
        plsc.subcore_barrier()


_sc_counts = pl.kernel(
    _sc_counts_body,
    out_type=(jax.ShapeDtypeStruct((NC, NP, D), jnp.float32),
              jax.ShapeDtypeStruct((NC, NP, D), jnp.float32)),
    mesh=_sc_mesh,
    scratch_types=(
        pltpu.VMEM_SHARED((NP, D), jnp.float32),
        pltpu.VMEM((CH,), jnp.int32),
        pltpu.VMEM((TAIL,), jnp.int32),
        pltpu.VMEM((ZCH, D), jnp.float32),
        pltpu.VMEM((CH, D), jnp.float32),
        pltpu.VMEM((TAIL, D), jnp.float32),
    ),
)


_RB = 1000
_G = N // _RB


def _mm_body(x_ref, w_ref, o_ref):
    o_ref[...] = jnp.dot(x_ref[...], w_ref[...],
                         preferred_element_type=jnp.float32)


def _tc_matmul(x, w):
    return pl.pallas_call(
        _mm_body,
        grid=(_G,),
        in_specs=[pl.BlockSpec((_RB, D), lambda i: (i, 0)),
                  pl.BlockSpec((D, D), lambda i: (0, 0))],
        out_specs=pl.BlockSpec((_RB, D), lambda i: (i, 0)),
        out_shape=jax.ShapeDtypeStruct((N, D), jnp.float32),
    )(x, w)


def _inv_counts(c_ref):
    c = c_ref[0, :, 0:1] + c_ref[1, :, 0:1]
    return jnp.where(c > 0, 1.0 / c, 0.0)


def _comb_body(p_ref, c_ref, o_ref):
    o_ref[...] = (p_ref[0] + p_ref[1]) * _inv_counts(c_ref)


def _tc_combine(p, c):
    return pl.pallas_call(
        _comb_body,
        grid=(_G,),
        in_specs=[pl.BlockSpec((NC, _RB, D), lambda i: (0, i, 0)),
                  pl.BlockSpec((NC, _RB, D), lambda i: (0, i, 0))],
        out_specs=pl.BlockSpec((_RB, D), lambda i: (i, 0)),
        out_shape=jax.ShapeDtypeStruct((N, D), jnp.float32),
    )(p, c)


def _mid_body(p_ref, c_ref, b_ref, g_ref, be_ref, w_ref, o_ref):
    h = (p_ref[0] + p_ref[1]) * _inv_counts(c_ref) + b_ref[...]
    mu = jnp.mean(h, axis=-1, keepdims=True)
    var = jnp.mean((h - mu) * (h - mu), axis=-1, keepdims=True)
    h = (h - mu) * lax.rsqrt(var + 1e-5) * g_ref[...] + be_ref[...]
    h = jnp.where(h >= 0, h, 0.01 * h)
    o_ref[...] = jnp.dot(h, w_ref[...], preferred_element_type=jnp.float32)


def _tc_mid(p, c, b1, gamma, beta, w3):
    vec = pl.BlockSpec((1, D), lambda i: (0, 0))
    return pl.pallas_call(
        _mid_body,
        grid=(_G,),
        in_specs=[pl.BlockSpec((NC, _RB, D), lambda i: (0, i, 0)),
                  pl.BlockSpec((NC, _RB, D), lambda i: (0, i, 0)),
                  vec, vec, vec,
                  pl.BlockSpec((D, D), lambda i: (0, 0))],
        out_specs=pl.BlockSpec((_RB, D), lambda i: (i, 0)),
        out_shape=jax.ShapeDtypeStruct((N, D), jnp.float32),
    )(p, c, b1.reshape(1, D), gamma.reshape(1, D), beta.reshape(1, D), w3)


def _final_body(p_ref, c_ref, b_ref, o_ref):
    o_ref[...] = (p_ref[0] + p_ref[1]) * _inv_counts(c_ref) + b_ref[...]


def _tc_final(p, c, b3):
    return pl.pallas_call(
        _final_body,
        grid=(_G,),
        in_specs=[pl.BlockSpec((NC, _RB, D), lambda i: (0, i, 0)),
                  pl.BlockSpec((NC, _RB, D), lambda i: (0, i, 0)),
                  pl.BlockSpec((1, D), lambda i: (0, 0))],
        out_specs=pl.BlockSpec((_RB, D), lambda i: (i, 0)),
        out_shape=jax.ShapeDtypeStruct((N, D), jnp.float32),
    )(p, c, b3.reshape(1, D))


@jax.jit
def kernel(x, hyperedge_index, W1, b1, gamma, beta, W3, b3):
    node_idx = hyperedge_index[0]
    he_idx = hyperedge_index[1]

    cnt_node, cnt_he = _sc_counts(node_idx, he_idx)
    x1 = _tc_matmul(x, W1)
    e_p = _sc_stage(x1, node_idx, he_idx)
    e1 = _tc_combine(e_p, cnt_he)
    o_p = _sc_stage(e1, he_idx, node_idx)
    x2 = _tc_mid(o_p, cnt_node, b1, gamma, beta, W3)
    e2_p = _sc_stage(x2, node_idx, he_idx)
    e2 = _tc_combine(e2_p, cnt_he)
    o2_p = _sc_stage(e2, he_idx, node_idx)
    return _tc_final(o2_p, cnt_node, b3)
